# baseline (speedup 1.0000x reference)
import functools
import math

import jax
import jax.numpy as jnp
from jax import lax
from jax.experimental import pallas as pl
from jax.experimental.pallas import tpu as pltpu
from jax.experimental.pallas import tpu_sc as plsc

N = 100000
NP = 100352
NPW = NP // 32
E = 1600000
EROWS = 12544
EROWS_A = EROWS + 16
EPAD = EROWS_A * 128
NC, NS = 2, 16
NTILES = NC * NS
DROWS_T = EROWS // NTILES
AROWS_T = EROWS // NS
NPT = NP // NS
K1, K2, K3 = 60000, 36000, 21600

_mesh = plsc.VectorSubcoreMesh(core_axis_name="c", subcore_axis_name="s")
_cparams = pltpu.CompilerParams(needs_layout_passes=False,
                                use_tc_tiling_on_sc=False)


def _wid(c, s):
    return c * NS + s


def _deg_body(srcr_hbm, dstr_hbm, actw_hbm, out_hbm, src_v, dst_v, actw_v, deg_v):
    c = lax.axis_index("c")
    s = lax.axis_index("s")
    wid = _wid(c, s)
    zero16 = jnp.zeros((16,), jnp.float32)

    def zbody(i, carry):
        for u in range(16):
            deg_v[pl.ds(i * 256 + u * 16, 16)] = zero16
        return carry

    lax.fori_loop(0, NP // 256, zbody, 0)
    pltpu.sync_copy(actw_hbm, actw_v)
    row0 = wid * DROWS_T

    def body(g, carry):
        r = row0 + g * 8
        pltpu.sync_copy(srcr_hbm.at[pl.ds(r, 8)], src_v)
        pltpu.sync_copy(dstr_hbm.at[pl.ds(r, 8)], dst_v)
        for j in range(8):
            for t in range(8):
                sl = pl.ds(t * 16, 16)
                sv = src_v[j, sl]
                dv = dst_v[j, sl]
                w = plsc.load_gather(actw_v, [jnp.right_shift(sv, 5)])
                bit = jnp.bitwise_and(
                    jnp.right_shift(w, jnp.bitwise_and(sv, 31)), 1)
                plsc.addupdate_scatter(deg_v, [dv], bit.astype(jnp.float32))
        return carry

    lax.fori_loop(0, DROWS_T // 8, body, 0)
    pltpu.sync_copy(deg_v, out_hbm.at[wid])


_deg_call = pl.kernel(
    _deg_body,
    out_type=jax.ShapeDtypeStruct((NTILES, NP), jnp.float32),
    mesh=_mesh,
    compiler_params=_cparams,
    scratch_types=[
        pltpu.VMEM((8, 128), jnp.int32),
        pltpu.VMEM((8, 128), jnp.int32),
        pltpu.VMEM((NPW,), jnp.int32),
        pltpu.VMEM((NP,), jnp.float32),
    ],
)


_AG = 4
_ARB = _AG * 128
_NPH = AROWS_T // _AG


def _agg_body(srcr_hbm, dstr_hbm, g2_hbm, zer_hbm, out_hbm,
              src_v, dst_v, dst2_v, idx_v, rows_v, sem_e, sem_g, sh):
    c = lax.axis_index("c")
    s = lax.axis_index("s")
    coff = c * NP
    pltpu.sync_copy(zer_hbm, sh.at[pl.ds(s * NPT, NPT)])
    plsc.subcore_barrier()
    row0 = s * AROWS_T

    def phase(p, b, do_scatter):
        r = row0 + (p + 1) * _AG
        e1 = pltpu.async_copy(srcr_hbm.at[pl.ds(r, _AG)], src_v.at[1 - b],
                              sem_e)
        e2 = pltpu.async_copy(dstr_hbm.at[pl.ds(r, _AG)], dst_v.at[1 - b],
                              sem_e)
        for j in range(_AG):
            for t in range(8):
                sl = pl.ds(t * 16, 16)
                idx_v[j, sl] = src_v[b, j, sl] + coff
        gd = [pltpu.async_copy(g2_hbm.at[idx_v.at[j]],
                               rows_v.at[b].at[pl.ds(j * 128, 128)], sem_g)
              for j in range(_AG)]
        for j in range(_AG):
            for t in range(8):
                sl = pl.ds(t * 16, 16)
                dst2_v[b, j, sl] = dst_v[b, j, sl]
        if do_scatter:
            for j in range(_AG):
                pltpu.sync_copy(rows_v.at[1 - b].at[pl.ds(j * 128, 128)],
                                sh.at[dst2_v.at[1 - b].at[j]], add=True)
        for d in gd:
            d.wait()
        e1.wait()
        e2.wait()

    def scatter_only(b):
        for j in range(_AG):
            pltpu.sync_copy(rows_v.at[b].at[pl.ds(j * 128, 128)],
                            sh.at[dst2_v.at[b].at[j]], add=True)

    pltpu.sync_copy(srcr_hbm.at[pl.ds(row0, _AG)], src_v.at[0])
    pltpu.sync_copy(dstr_hbm.at[pl.ds(row0, _AG)], dst_v.at[0])
    phase(0, 0, False)

    def body(i, carry):
        phase(2 * i + 1, 1, True)
        phase(2 * i + 2, 0, True)
        return carry

    lax.fori_loop(0, (_NPH - 2) // 2, body, 0)
    phase(_NPH - 1, 1, True)
    scatter_only(1)
    plsc.subcore_barrier()
    pltpu.sync_copy(sh.at[pl.ds(s * NPT, NPT)],
                    out_hbm.at[pl.ds(coff + s * NPT, NPT)])


_agg_call = pl.kernel(
    _agg_body,
    out_type=jax.ShapeDtypeStruct((NC * NP, 16), jnp.float32),
    mesh=_mesh,
    compiler_params=_cparams,
    scratch_types=[
        pltpu.VMEM((2, _AG, 128), jnp.int32),
        pltpu.VMEM((2, _AG, 128), jnp.int32),
        pltpu.VMEM((2, _AG, 128), jnp.int32),
        pltpu.VMEM((_AG, 128), jnp.int32),
        pltpu.VMEM((2, _ARB, 16), jnp.float32),
        pltpu.SemaphoreType.DMA,
        pltpu.SemaphoreType.DMA,
        pltpu.VMEM_SHARED((NP, 16), jnp.float32),
    ],
)


def _hist_body(shift_lo, shift_hi, nb, check_prefix,
               m_hbm, par_hbm, out_hbm, m_v, par_s, hist_v):
    c = lax.axis_index("c")
    s = lax.axis_index("s")
    wid = _wid(c, s)
    zero16 = jnp.zeros((16,), jnp.float32)
    one16 = jnp.ones((16,), jnp.float32)

    def zb(i, carry):
        for u in range(16):
            hist_v[pl.ds(i * 256 + u * 16, 16)] = zero16
        return carry

    lax.fori_loop(0, nb // 256, zb, 0)
    pltpu.sync_copy(m_hbm.at[pl.ds(wid * NPW, NPW)], m_v)
    pltpu.sync_copy(par_hbm, par_s)
    prefix = par_s[...].astype(jnp.uint32)
    maskv = jnp.full((16,), nb - 1, jnp.uint32)

    def body(i, carry):
        for u in range(4):
            mv = m_v[pl.ds(i * 64 + u * 16, 16)]
            key = jnp.bitwise_and(jnp.right_shift(mv, jnp.uint32(shift_lo)),
                                  maskv)
            binv = key.astype(jnp.int32)
            if check_prefix:
                pm = jnp.right_shift(mv, jnp.uint32(shift_hi)) == prefix
                plsc.addupdate_scatter(hist_v, [binv], one16, mask=pm)
            else:
                plsc.addupdate_scatter(hist_v, [binv], one16)
        return carry

    lax.fori_loop(0, NPW // 64, body, 0)
    pltpu.sync_copy(hist_v, out_hbm.at[wid])


def _make_hist(shift_lo, shift_hi, nb, check_prefix):
    return pl.kernel(
        functools.partial(_hist_body, shift_lo, shift_hi, nb, check_prefix),
        out_type=jax.ShapeDtypeStruct((NTILES, nb), jnp.float32),
        mesh=_mesh,
        compiler_params=_cparams,
        scratch_types=[
            pltpu.VMEM((NPW,), jnp.uint32),
            pltpu.VMEM((16,), jnp.int32),
            pltpu.VMEM((nb,), jnp.float32),
        ],
    )


_hist1 = _make_hist(16, 0, 65536, False)
_hist2 = _make_hist(0, 16, 65536, True)


def _pick(hist, k_rem, nb):
    csum = jnp.cumsum(hist[::-1])[::-1]
    ge = csum >= k_rem
    b = jnp.max(jnp.where(ge, jnp.arange(nb), -1))
    above = jnp.where(b + 1 < nb, csum[jnp.minimum(b + 1, nb - 1)], 0.0)
    return b, above


def _conv(hWT, actf, actw, b, srcr, dstr, zeros16):
    degp = _deg_call(srcr, dstr, actw)
    deg = jnp.sum(degp, axis=0) + 1.0
    dinv = lax.rsqrt(deg)
    gT = hWT * (dinv * actf)[None, :]
    g2 = jnp.concatenate([gT[:16].T, gT[16:].T], axis=0)
    A2 = _agg_call(srcr, dstr, g2, zeros16)
    AT = jnp.concatenate([A2[:NP].T, A2[NP:].T], axis=0)
    out = dinv[None, :] * AT + hWT * (1.0 / deg)[None, :] + b[:, None]
    return jax.nn.relu(out)


def _pool(hT, actf, p, k):
    u = jnp.dot(p / jnp.linalg.norm(p), hT)
    bits = lax.bitcast_convert_type(u.astype(jnp.float32), jnp.uint32)
    m = jnp.where(bits >> 31 == 1, ~bits, bits | jnp.uint32(0x80000000))
    m = jnp.where(actf > 0, m, jnp.uint32(0))

    zpar = jnp.zeros((16,), jnp.int32)
    h1 = jnp.sum(_hist1(m, zpar), axis=0)
    b1, a1 = _pick(h1, jnp.float32(k), 65536)
    k2r = k - a1
    h2 = jnp.sum(_hist2(m, jnp.full((16,), b1, jnp.int32)), axis=0)
    b2, a2 = _pick(h2, k2r, 65536)
    r = k2r - a2
    T = (b1.astype(jnp.uint32) << 16) | b2.astype(jnp.uint32)
    gt = m > T
    eq = m == T
    cum = jnp.cumsum(eq.astype(jnp.int32)).astype(jnp.float32)
    sel = gt | (eq & (cum <= r))
    self_f = sel.astype(jnp.float32)
    h_new = hT * (jnp.tanh(u) * self_f)[None, :]
    selw = lax.bitcast_convert_type(
        jnp.sum(sel.reshape(NPW, 32).astype(jnp.uint32)
                << jnp.arange(32, dtype=jnp.uint32), axis=1), jnp.int32)
    return h_new, self_f, selw


def _readout(hT, actf, k):
    neg = jnp.where(actf[None, :] > 0, hT, -jnp.inf)
    mx = jnp.max(neg, axis=1)
    mn = jnp.dot(hT, actf) / k
    return jnp.concatenate([mx, mn])[None, :]


def kernel(x, edge_index, batch, W1, b1, W2, b2, W3, b3,
           p1, p2, p3, Wl1, bl1, Wl2, bl2):
    src = edge_index[0]
    dst = edge_index[1]
    padi = jnp.full((EPAD - E,), NP - 1, jnp.int32)
    srcr = jnp.concatenate([src, padi]).reshape(EROWS_A, 128)
    dstr = jnp.concatenate([dst, padi]).reshape(EROWS_A, 128)
    zeros16 = jnp.zeros((NPT, 16), jnp.float32)

    actf = jnp.concatenate([jnp.ones((N,), jnp.float32),
                            jnp.zeros((NP - N,), jnp.float32)])
    actw = jnp.concatenate([jnp.full((N // 32,), -1, jnp.int32),
                            jnp.zeros((NPW - N // 32,), jnp.int32)])

    xT = jnp.zeros((9, NP), jnp.float32).at[:, :N].set(x.T)

    h = _conv(W1.T @ xT, actf, actw, b1, srcr, dstr, zeros16)
    h, actf, actw = _pool(h, actf, p1, K1)
    x1 = _readout(h, actf, K1)
    h = _conv(W2.T @ h, actf, actw, b2, srcr, dstr, zeros16)
    h, actf, actw = _pool(h, actf, p2, K2)
    x2 = _readout(h, actf, K2)
    h = _conv(W3.T @ h, actf, actw, b3, srcr, dstr, zeros16)
    h, actf, actw = _pool(h, actf, p3, K3)
    x3 = _readout(h, actf, K3)

    z = jnp.concatenate([x1, x2, x3], axis=1)
    z = jax.nn.relu(z @ Wl1 + bl1)
    z = jax.nn.relu(z @ Wl2 + bl2)
    return jnp.concatenate([jax.nn.softplus(z[:, 0:3]),
                            jax.nn.sigmoid(z[:, 3:10])], axis=1)


# speedup vs baseline: 1.2198x; 1.2198x over previous
import functools
import math

import jax
import jax.numpy as jnp
from jax import lax
from jax.experimental import pallas as pl
from jax.experimental.pallas import tpu as pltpu
from jax.experimental.pallas import tpu_sc as plsc

N = 100000
NP = 100352
NPW = NP // 32
E = 1600000
EROWS = 12544
EROWS_A = EROWS + 16
EPAD = EROWS_A * 128
NC, NS = 2, 16
NTILES = NC * NS
DROWS_T = EROWS // NTILES
AROWS_T = EROWS // NS
NPT = NP // NS
K1, K2, K3 = 60000, 36000, 21600

_mesh = plsc.VectorSubcoreMesh(core_axis_name="c", subcore_axis_name="s")
_cparams = pltpu.CompilerParams(needs_layout_passes=False,
                                use_tc_tiling_on_sc=False)


def _wid(c, s):
    return c * NS + s


def _deg_body(srcr_hbm, dstr_hbm, actw_hbm, out_hbm, src_v, dst_v, actw_v, deg_v):
    c = lax.axis_index("c")
    s = lax.axis_index("s")
    wid = _wid(c, s)
    zero16 = jnp.zeros((16,), jnp.float32)

    def zbody(i, carry):
        for u in range(16):
            deg_v[pl.ds(i * 256 + u * 16, 16)] = zero16
        return carry

    lax.fori_loop(0, NP // 256, zbody, 0)
    pltpu.sync_copy(actw_hbm, actw_v)
    row0 = wid * DROWS_T

    def body(g, carry):
        r = row0 + g * 8
        pltpu.sync_copy(srcr_hbm.at[pl.ds(r, 8)], src_v)
        pltpu.sync_copy(dstr_hbm.at[pl.ds(r, 8)], dst_v)
        for j in range(8):
            for t in range(8):
                sl = pl.ds(t * 16, 16)
                sv = src_v[j, sl]
                dv = dst_v[j, sl]
                w = plsc.load_gather(actw_v, [jnp.right_shift(sv, 5)])
                bit = jnp.bitwise_and(
                    jnp.right_shift(w, jnp.bitwise_and(sv, 31)), 1)
                plsc.addupdate_scatter(deg_v, [dv], bit.astype(jnp.float32))
        return carry

    lax.fori_loop(0, DROWS_T // 8, body, 0)
    pltpu.sync_copy(deg_v, out_hbm.at[wid])


_deg_call = pl.kernel(
    _deg_body,
    out_type=jax.ShapeDtypeStruct((NTILES, NP), jnp.float32),
    mesh=_mesh,
    compiler_params=_cparams,
    scratch_types=[
        pltpu.VMEM((8, 128), jnp.int32),
        pltpu.VMEM((8, 128), jnp.int32),
        pltpu.VMEM((NPW,), jnp.int32),
        pltpu.VMEM((NP,), jnp.float32),
    ],
)


_AG = 4
_ARB = _AG * 128
_NPH = AROWS_T // _AG


def _agg_body(srcr_hbm, dstr_hbm, g2_hbm, zer_hbm, out_hbm,
              src_v, dst_v, dst2_v, idx_v, rows_v, sem_e, sem_g, sh):
    c = lax.axis_index("c")
    s = lax.axis_index("s")
    coff = c * NP
    pltpu.sync_copy(zer_hbm, sh.at[pl.ds(s * NPT, NPT)])
    plsc.subcore_barrier()
    row0 = s * AROWS_T

    def phase(p, b, do_scatter):
        r = row0 + (p + 1) * _AG
        e1 = pltpu.async_copy(srcr_hbm.at[pl.ds(r, _AG)], src_v.at[1 - b],
                              sem_e)
        e2 = pltpu.async_copy(dstr_hbm.at[pl.ds(r, _AG)], dst_v.at[1 - b],
                              sem_e)
        for j in range(_AG):
            for t in range(8):
                sl = pl.ds(t * 16, 16)
                idx_v[j, sl] = src_v[b, j, sl] + coff
        gd = [pltpu.async_copy(g2_hbm.at[idx_v.at[j]],
                               rows_v.at[b].at[pl.ds(j * 128, 128)], sem_g)
              for j in range(_AG)]
        for j in range(_AG):
            for t in range(8):
                sl = pl.ds(t * 16, 16)
                dst2_v[b, j, sl] = dst_v[b, j, sl]
        if do_scatter:
            for j in range(_AG):
                pltpu.sync_copy(rows_v.at[1 - b].at[pl.ds(j * 128, 128)],
                                sh.at[dst2_v.at[1 - b].at[j]], add=True)
        for d in gd:
            d.wait()
        e1.wait()
        e2.wait()

    def scatter_only(b):
        for j in range(_AG):
            pltpu.sync_copy(rows_v.at[b].at[pl.ds(j * 128, 128)],
                            sh.at[dst2_v.at[b].at[j]], add=True)

    pltpu.sync_copy(srcr_hbm.at[pl.ds(row0, _AG)], src_v.at[0])
    pltpu.sync_copy(dstr_hbm.at[pl.ds(row0, _AG)], dst_v.at[0])
    phase(0, 0, False)

    def body(i, carry):
        phase(2 * i + 1, 1, True)
        phase(2 * i + 2, 0, True)
        return carry

    lax.fori_loop(0, (_NPH - 2) // 2, body, 0)
    phase(_NPH - 1, 1, True)
    scatter_only(1)
    plsc.subcore_barrier()
    pltpu.sync_copy(sh.at[pl.ds(s * NPT, NPT)],
                    out_hbm.at[pl.ds(coff + s * NPT, NPT)])


_agg_call = pl.kernel(
    _agg_body,
    out_type=jax.ShapeDtypeStruct((NC * NP, 16), jnp.float32),
    mesh=_mesh,
    compiler_params=_cparams,
    scratch_types=[
        pltpu.VMEM((2, _AG, 128), jnp.int32),
        pltpu.VMEM((2, _AG, 128), jnp.int32),
        pltpu.VMEM((2, _AG, 128), jnp.int32),
        pltpu.VMEM((_AG, 128), jnp.int32),
        pltpu.VMEM((2, _ARB, 16), jnp.float32),
        pltpu.SemaphoreType.DMA,
        pltpu.SemaphoreType.DMA,
        pltpu.VMEM_SHARED((NP, 16), jnp.float32),
    ],
)


def _hist_body(shift_lo, shift_hi, nb, check_prefix,
               m_hbm, par_hbm, out_hbm, m_v, par_s, hist_v):
    c = lax.axis_index("c")
    s = lax.axis_index("s")
    wid = _wid(c, s)
    zero16 = jnp.zeros((16,), jnp.float32)
    one16 = jnp.ones((16,), jnp.float32)

    def zb(i, carry):
        for u in range(16):
            hist_v[pl.ds(i * 256 + u * 16, 16)] = zero16
        return carry

    lax.fori_loop(0, nb // 256, zb, 0)
    pltpu.sync_copy(m_hbm.at[pl.ds(wid * NPW, NPW)], m_v)
    pltpu.sync_copy(par_hbm, par_s)
    prefix = par_s[...].astype(jnp.uint32)
    maskv = jnp.full((16,), nb - 1, jnp.uint32)

    def body(i, carry):
        for u in range(4):
            mv = m_v[pl.ds(i * 64 + u * 16, 16)]
            key = jnp.bitwise_and(jnp.right_shift(mv, jnp.uint32(shift_lo)),
                                  maskv)
            binv = key.astype(jnp.int32)
            if check_prefix:
                pm = jnp.right_shift(mv, jnp.uint32(shift_hi)) == prefix
                plsc.addupdate_scatter(hist_v, [binv], one16, mask=pm)
            else:
                plsc.addupdate_scatter(hist_v, [binv], one16)
        return carry

    lax.fori_loop(0, NPW // 64, body, 0)
    pltpu.sync_copy(hist_v, out_hbm.at[wid])


def _make_hist(shift_lo, shift_hi, nb, check_prefix):
    return pl.kernel(
        functools.partial(_hist_body, shift_lo, shift_hi, nb, check_prefix),
        out_type=jax.ShapeDtypeStruct((NTILES, nb), jnp.float32),
        mesh=_mesh,
        compiler_params=_cparams,
        scratch_types=[
            pltpu.VMEM((NPW,), jnp.uint32),
            pltpu.VMEM((16,), jnp.int32),
            pltpu.VMEM((nb,), jnp.float32),
        ],
    )


_hist1 = _make_hist(21, 0, 2048, False)
_hist2 = _make_hist(10, 21, 2048, True)
_hist3 = _make_hist(0, 10, 1024, True)


def _pick(hist, k_rem, nb):
    csum = jnp.cumsum(hist[::-1])[::-1]
    ge = csum >= k_rem
    b = jnp.max(jnp.where(ge, jnp.arange(nb), -1))
    above = jnp.where(b + 1 < nb, csum[jnp.minimum(b + 1, nb - 1)], 0.0)
    return b, above


def _conv(hWT, actf, actw, b, srcr, dstr, zeros16):
    degp = _deg_call(srcr, dstr, actw)
    deg = jnp.sum(degp, axis=0) + 1.0
    dinv = lax.rsqrt(deg)
    gT = hWT * (dinv * actf)[None, :]
    g2 = gT.reshape(2, 16, NP).transpose(0, 2, 1).reshape(2 * NP, 16)
    A2 = _agg_call(srcr, dstr, g2, zeros16)
    AT = A2.reshape(2, NP, 16).transpose(0, 2, 1).reshape(32, NP)
    out = dinv[None, :] * AT + hWT * (1.0 / deg)[None, :] + b[:, None]
    return jax.nn.relu(out)


def _pool(hT, actf, p, k):
    u = jnp.dot(p / jnp.linalg.norm(p), hT)
    bits = lax.bitcast_convert_type(u.astype(jnp.float32), jnp.uint32)
    m = jnp.where(bits >> 31 == 1, ~bits, bits | jnp.uint32(0x80000000))
    m = jnp.where(actf > 0, m, jnp.uint32(0))

    zpar = jnp.zeros((16,), jnp.int32)
    h1 = jnp.sum(_hist1(m, zpar), axis=0)
    b1, a1 = _pick(h1, jnp.float32(k), 2048)
    k2r = k - a1
    h2 = jnp.sum(_hist2(m, jnp.full((16,), b1, jnp.int32)), axis=0)
    b2, a2 = _pick(h2, k2r, 2048)
    k3r = k2r - a2
    pref3 = (b1 << 11) | b2
    h3 = jnp.sum(_hist3(m, jnp.full((16,), pref3, jnp.int32)), axis=0)
    b3, a3 = _pick(h3, k3r, 1024)
    r = k3r - a3
    T = ((b1.astype(jnp.uint32) << 21) | (b2.astype(jnp.uint32) << 10)
         | b3.astype(jnp.uint32))
    gt = m > T
    eq = m == T
    cum = jnp.cumsum(eq.astype(jnp.int32)).astype(jnp.float32)
    sel = gt | (eq & (cum <= r))
    self_f = sel.astype(jnp.float32)
    h_new = hT * (jnp.tanh(u) * self_f)[None, :]
    selw = lax.bitcast_convert_type(
        jnp.sum(sel.reshape(NPW, 32).astype(jnp.uint32)
                << jnp.arange(32, dtype=jnp.uint32), axis=1), jnp.int32)
    return h_new, self_f, selw


def _readout(hT, actf, k):
    neg = jnp.where(actf[None, :] > 0, hT, -jnp.inf)
    mx = jnp.max(neg, axis=1)
    mn = jnp.dot(hT, actf) / k
    return jnp.concatenate([mx, mn])[None, :]


def kernel(x, edge_index, batch, W1, b1, W2, b2, W3, b3,
           p1, p2, p3, Wl1, bl1, Wl2, bl2):
    src = edge_index[0]
    dst = edge_index[1]
    padi = jnp.full((EPAD - E,), NP - 1, jnp.int32)
    srcr = jnp.concatenate([src, padi]).reshape(EROWS_A, 128)
    dstr = jnp.concatenate([dst, padi]).reshape(EROWS_A, 128)
    zeros16 = jnp.zeros((NPT, 16), jnp.float32)

    actf = jnp.concatenate([jnp.ones((N,), jnp.float32),
                            jnp.zeros((NP - N,), jnp.float32)])
    actw = jnp.concatenate([jnp.full((N // 32,), -1, jnp.int32),
                            jnp.zeros((NPW - N // 32,), jnp.int32)])

    xT = jnp.zeros((9, NP), jnp.float32).at[:, :N].set(x.T)

    h = _conv(W1.T @ xT, actf, actw, b1, srcr, dstr, zeros16)
    h, actf, actw = _pool(h, actf, p1, K1)
    x1 = _readout(h, actf, K1)
    h = _conv(W2.T @ h, actf, actw, b2, srcr, dstr, zeros16)
    h, actf, actw = _pool(h, actf, p2, K2)
    x2 = _readout(h, actf, K2)
    h = _conv(W3.T @ h, actf, actw, b3, srcr, dstr, zeros16)
    h, actf, actw = _pool(h, actf, p3, K3)
    x3 = _readout(h, actf, K3)

    z = jnp.concatenate([x1, x2, x3], axis=1)
    z = jax.nn.relu(z @ Wl1 + bl1)
    z = jax.nn.relu(z @ Wl2 + bl2)
    return jnp.concatenate([jax.nn.softplus(z[:, 0:3]),
                            jax.nn.sigmoid(z[:, 3:10])], axis=1)
